# Initial kernel scaffold; baseline (speedup 1.0000x reference)
import jax
import jax.numpy as jnp
from jax.experimental import pallas as pl


def kernel(x, pe):
    raise NotImplementedError("write your pallas kernel here")


# speedup vs baseline: 6.7425x; 6.7425x over previous
import functools

import jax
import jax.numpy as jnp
from jax import lax
from jax.experimental import pallas as pl
from jax.experimental.pallas import tpu as pltpu
from jax.experimental.pallas import tpu_sc as plsc


def _rel_pos_sc(pe, seq_len):
    table_rows, dim = pe.shape
    info = plsc.get_sparse_core_info()
    num_cores, num_subcores = info.num_cores, info.num_subcores
    num_workers = num_cores * num_subcores
    rows_per_worker = seq_len // num_workers

    mesh = plsc.VectorSubcoreMesh(core_axis_name="c", subcore_axis_name="s")

    @functools.partial(
        pl.kernel,
        mesh=mesh,
        out_type=jax.ShapeDtypeStruct((seq_len, seq_len, dim), jnp.float32),
        scratch_types=[pltpu.VMEM_SHARED((table_rows, dim), jnp.float32)],
    )
    def k(pe_hbm, out_hbm, pe_sh):
        c = lax.axis_index("c")
        s = lax.axis_index("s")

        @pl.when(s == 0)
        def _():
            pltpu.sync_copy(pe_hbm, pe_sh)

        plsc.subcore_barrier()

        base = (c * num_subcores + s) * rows_per_worker

        def body(r, carry):
            i = base + r
            start = (seq_len - 1) - i
            pltpu.sync_copy(
                pe_sh.at[pl.ds(start, seq_len), :],
                out_hbm.at[i],
            )
            return carry

        lax.fori_loop(0, rows_per_worker, body, 0)

    return k(pe)


def kernel(x, pe):
    seq_len = x.shape[2]
    return _rel_pos_sc(pe, seq_len)


# speedup vs baseline: 6.7912x; 1.0072x over previous
import functools

import jax
import jax.numpy as jnp
from jax import lax
from jax.experimental import pallas as pl
from jax.experimental.pallas import tpu as pltpu
from jax.experimental.pallas import tpu_sc as plsc


def _rel_pos_sc(pe, seq_len):
    table_rows, dim = pe.shape
    info = plsc.get_sparse_core_info()
    num_cores, num_subcores = info.num_cores, info.num_subcores
    num_workers = num_cores * num_subcores
    rows_per_worker = seq_len // num_workers

    mesh = plsc.VectorSubcoreMesh(core_axis_name="c", subcore_axis_name="s")

    fire = 16

    @functools.partial(
        pl.kernel,
        mesh=mesh,
        out_type=jax.ShapeDtypeStruct((seq_len, seq_len, dim), jnp.float32),
        scratch_types=[
            pltpu.VMEM_SHARED((table_rows, dim), jnp.float32),
            pltpu.SemaphoreType.DMA,
        ],
    )
    def k(pe_hbm, out_hbm, pe_sh, sem):
        c = lax.axis_index("c")
        s = lax.axis_index("s")

        @pl.when(s == 0)
        def _():
            pltpu.sync_copy(pe_hbm, pe_sh)

        plsc.subcore_barrier()

        base = (c * num_subcores + s) * rows_per_worker

        def copy_desc(i):
            start = (seq_len - 1) - i
            return pltpu.make_async_copy(
                pe_sh.at[pl.ds(start, seq_len), :], out_hbm.at[i], sem
            )

        def chunk(ci, carry):
            row0 = base + ci * fire
            for b in range(fire):
                copy_desc(row0 + b).start()
            for b in range(fire):
                copy_desc(row0 + b).wait()
            return carry

        lax.fori_loop(0, rows_per_worker // fire, chunk, 0)

    return k(pe)


def kernel(x, pe):
    seq_len = x.shape[2]
    return _rel_pos_sc(pe, seq_len)
